# baseline (speedup 1.0000x reference)
import jax
import jax.numpy as jnp
from jax import lax
from jax.experimental import pallas as pl
from jax.experimental.pallas import tpu as pltpu
from jax.experimental.pallas import tpu_sc as plsc

BATCH = 16384
EMBED = 32
HIDDEN = 64
PADW = 128

NUM_CORES = 2
NUM_SUBCORES = 16
NUM_WORKERS = NUM_CORES * NUM_SUBCORES
B_PER_W = BATCH // NUM_WORKERS

CHUNK = 16


def _gather_rows(tab_hbm, idx_s, out_hbm, out_base, sem):
    def chunk_body(c):
        base = c * CHUNK
        idx_vec = idx_s[pl.ds(base, CHUNK)]
        copies = []
        for j in range(CHUNK):
            r = idx_vec[j]
            copies.append(
                pltpu.async_copy(tab_hbm.at[pl.ds(r, 1)],
                                 out_hbm.at[pl.ds(out_base + base + j, 1)],
                                 sem))
        for cp in copies:
            cp.wait()

    pl.loop(0, B_PER_W // CHUNK)(chunk_body)


def _sc_gather_body(user_hbm, item_hbm, ut_hbm, it_hbm, ue_out, ie_out,
                    uidx_v, iidx_v, sem_u, sem_i):
    wid = lax.axis_index("s") * NUM_CORES + lax.axis_index("c")
    base = wid * B_PER_W
    pltpu.sync_copy(user_hbm.at[pl.ds(base, B_PER_W)], uidx_v)
    pltpu.sync_copy(item_hbm.at[pl.ds(base, B_PER_W)], iidx_v)
    _gather_rows(ut_hbm, uidx_v, ue_out, base, sem_u)
    _gather_rows(it_hbm, iidx_v, ie_out, base, sem_i)


def _sc_gather(user, item, user_table, item_table):
    mesh = plsc.VectorSubcoreMesh(core_axis_name="c", subcore_axis_name="s")
    return pl.kernel(
        _sc_gather_body,
        out_type=[
            jax.ShapeDtypeStruct((BATCH, EMBED), jnp.float32),
            jax.ShapeDtypeStruct((BATCH, EMBED), jnp.float32),
        ],
        mesh=mesh,
        scratch_types=[
            pltpu.VMEM((B_PER_W,), jnp.int32),
            pltpu.VMEM((B_PER_W,), jnp.int32),
            pltpu.SemaphoreType.DMA,
            pltpu.SemaphoreType.DMA,
        ],
    )(user, item, user_table, item_table)


TB = 2048


def _mlp_body(ue_ref, ie_ref, w1u_ref, w1i_ref, b1_ref, w2_ref, b2_ref, out_ref):
    h = jnp.dot(ue_ref[...], w1u_ref[...], preferred_element_type=jnp.float32)
    h = h + jnp.dot(ie_ref[...], w1i_ref[...], preferred_element_type=jnp.float32)
    h = jnp.maximum(h + b1_ref[...], 0.0)
    out_ref[...] = jnp.sum(h * w2_ref[...], axis=1, keepdims=True) + b2_ref[...]


def _tc_mlp(ue, ie, W1, b1, W2, b2):
    w1u = W1[:EMBED]
    w1i = W1[EMBED:]
    b1r = b1.reshape(1, HIDDEN)
    w2r = W2.reshape(1, HIDDEN)
    b2r = b2.reshape(1, 1)
    grid = (BATCH // TB,)
    return pl.pallas_call(
        _mlp_body,
        grid=grid,
        in_specs=[
            pl.BlockSpec((TB, EMBED), lambda i: (i, 0)),
            pl.BlockSpec((TB, EMBED), lambda i: (i, 0)),
            pl.BlockSpec((EMBED, HIDDEN), lambda i: (0, 0)),
            pl.BlockSpec((EMBED, HIDDEN), lambda i: (0, 0)),
            pl.BlockSpec((1, HIDDEN), lambda i: (0, 0)),
            pl.BlockSpec((1, HIDDEN), lambda i: (0, 0)),
            pl.BlockSpec((1, 1), lambda i: (0, 0)),
        ],
        out_specs=pl.BlockSpec((TB, 1), lambda i: (i, 0)),
        out_shape=jax.ShapeDtypeStruct((BATCH, 1), jnp.float32),
    )(ue, ie, w1u, w1i, b1r, w2r, b2r)


@jax.jit
def kernel(user, item, user_table, item_table, W1, b1, W2, b2):
    ue, ie = _sc_gather(user, item, user_table, item_table)
    return _tc_mlp(ue, ie, W1, b1, W2, b2)


# speedup vs baseline: 5.3641x; 5.3641x over previous
import jax
import jax.numpy as jnp
from jax import lax
from jax.experimental import pallas as pl
from jax.experimental.pallas import tpu as pltpu
from jax.experimental.pallas import tpu_sc as plsc

BATCH = 16384
EMBED = 32
HIDDEN = 64
NUSER = 200000
NITEM = 30000

NUM_CORES = 2
NUM_SUBCORES = 16
NUM_WORKERS = NUM_CORES * NUM_SUBCORES
B_PER_W = BATCH // NUM_WORKERS

N_CHUNKS = 4
FPC = EMBED // N_CHUNKS


def _make_gather_body(nrows, nfeat):
    def body(idx_hbm, tab_hbm, out_hbm, idx_v, rows_v, sem):
        wid = lax.axis_index("s") * NUM_CORES + lax.axis_index("c")
        base = wid * B_PER_W
        pltpu.sync_copy(idx_hbm.at[pl.ds(base, B_PER_W)], idx_v)
        copies = []
        for j in range(nfeat):
            copies.append(pltpu.async_copy(
                tab_hbm.at[pl.ds(j * nrows, nrows)].at[idx_v],
                rows_v.at[j], sem))
        for cp in copies:
            cp.wait()
        pltpu.sync_copy(rows_v, out_hbm.at[:, pl.ds(base, B_PER_W)])

    return body


def _sc_gather_one(idx, tab_flat, nrows, nfeat):
    mesh = plsc.VectorSubcoreMesh(core_axis_name="c", subcore_axis_name="s")
    return pl.kernel(
        _make_gather_body(nrows, nfeat),
        out_type=jax.ShapeDtypeStruct((nfeat, BATCH), jnp.float32),
        mesh=mesh,
        scratch_types=[
            pltpu.VMEM((B_PER_W,), jnp.int32),
            pltpu.VMEM((nfeat, B_PER_W), jnp.float32),
            pltpu.SemaphoreType.DMA,
        ],
        compiler_params=pltpu.CompilerParams(use_tc_tiling_on_sc=False),
    )(idx, tab_flat)


TB = 2048


def _mlp_body(*refs):
    ue_refs = refs[:N_CHUNKS]
    ieT_ref = refs[N_CHUNKS]
    w1u_refs = refs[N_CHUNKS + 1:2 * N_CHUNKS + 1]
    w1i_ref, b1_ref, w2_ref, b2_ref, out_ref = refs[2 * N_CHUNKS + 1:]
    dn = (((0,), (0,)), ((), ()))
    hT = lax.dot_general(w1i_ref[...], ieT_ref[...], dn,
                         preferred_element_type=jnp.float32)
    for c in range(N_CHUNKS):
        hT = hT + lax.dot_general(w1u_refs[c][...], ue_refs[c][...], dn,
                                  preferred_element_type=jnp.float32)
    hT = jnp.maximum(hT + b1_ref[...], 0.0)
    out_ref[...] = lax.dot_general(w2_ref[...], hT, dn,
                                   preferred_element_type=jnp.float32) + b2_ref[...]


def _tc_mlp(ue_chunks, ieT, W1, b1, W2, b2):
    w1u = [W1[c * FPC:(c + 1) * FPC] for c in range(N_CHUNKS)]
    w1i = W1[EMBED:]
    b1r = b1.reshape(HIDDEN, 1)
    b2r = b2.reshape(1, 1)
    grid = (BATCH // TB,)
    in_specs = (
        [pl.BlockSpec((FPC, TB), lambda i: (0, i)) for _ in range(N_CHUNKS)]
        + [pl.BlockSpec((EMBED, TB), lambda i: (0, i))]
        + [pl.BlockSpec((FPC, HIDDEN), lambda i: (0, 0)) for _ in range(N_CHUNKS)]
        + [
            pl.BlockSpec((EMBED, HIDDEN), lambda i: (0, 0)),
            pl.BlockSpec((HIDDEN, 1), lambda i: (0, 0)),
            pl.BlockSpec((HIDDEN, 1), lambda i: (0, 0)),
            pl.BlockSpec((1, 1), lambda i: (0, 0)),
        ]
    )
    return pl.pallas_call(
        _mlp_body,
        grid=grid,
        in_specs=in_specs,
        out_specs=pl.BlockSpec((1, TB), lambda i: (0, i)),
        out_shape=jax.ShapeDtypeStruct((1, BATCH), jnp.float32),
    )(*ue_chunks, ieT, *w1u, w1i, b1r, W2, b2r)


@jax.jit
def kernel(user, item, user_table, item_table, W1, b1, W2, b2):
    it_flat = item_table.T.reshape(-1)
    ut_flats = []
    prev = it_flat
    for c in range(N_CHUNKS):
        user_table, prev = lax.optimization_barrier((user_table, prev))
        prev = user_table[:, c * FPC:(c + 1) * FPC].T.reshape(-1)
        ut_flats.append(prev)
    ieT = _sc_gather_one(item, it_flat, NITEM, EMBED)
    ue_chunks = [_sc_gather_one(user, f, NUSER, FPC) for f in ut_flats]
    out_row = _tc_mlp(ue_chunks, ieT, W1, b1, W2, b2)
    return out_row.reshape(BATCH, 1)
